# baseline (speedup 1.0000x reference)
import jax
import jax.numpy as jnp
from jax import lax
from jax.experimental import pallas as pl
from jax.experimental.pallas import tpu as pltpu
from jax.experimental.pallas import tpu_sc as plsc

R, C = 128, 100000
L = 16
G = 4
VS = 28000
VW = 400
NCH = 5
WIN = VW * NCH
SHARD = VS // 16
BV = 400
NEG_INF = float("-inf")


def _merge(ma, ia, mb, ib):
    take = (mb > ma) | ((mb == ma) & (ib < ia))
    return jnp.where(take, mb, ma), jnp.where(take, ib, ia)


def _sc_argmax(xt):
    mesh = plsc.VectorSubcoreMesh(core_axis_name="c", subcore_axis_name="s")

    @pl.kernel(
        out_type=(jax.ShapeDtypeStruct((R,), jnp.int32),
                  jax.ShapeDtypeStruct((R,), jnp.float32),
                  jax.ShapeDtypeStruct((2048,), jnp.float32),
                  jax.ShapeDtypeStruct((2048,), jnp.int32)),
        mesh=mesh,
        scratch_types=[
            pltpu.VMEM((VW, 128), jnp.float32),
            pltpu.VMEM((VW, 128), jnp.float32),
            pltpu.VMEM((G, L), jnp.float32),
            pltpu.VMEM((G, L), jnp.int32),
            pltpu.VMEM((L * L,), jnp.float32),
            pltpu.VMEM((L * L,), jnp.int32),
            pltpu.VMEM((L,), jnp.int32),
            pltpu.VMEM((L,), jnp.float32),
            pltpu.SemaphoreType.DMA,
            pltpu.SemaphoreType.DMA,
            pltpu.SemaphoreType.DMA,
        ],
    )
    def body(x_hbm, out_hbm, outm_hbm, xchm_hbm, xchc_hbm,
             bufA, bufB, mv, iv, gm, gi, res_v, resm_v, semA, semB, semC):
        s = lax.axis_index("s")
        c = lax.axis_index("c")
        bufs = (bufA, bufB)
        sems = (semA, semB)
        off0 = (s * SHARD) // 8 * 8
        off0 = pl.multiple_of(jnp.minimum(off0, VS - WIN), 8)

        def start(j):
            return pltpu.async_copy(
                x_hbm.at[pl.ds(pl.multiple_of(off0 + j * VW, 8), VW)],
                bufs[j % 2], sems[j % 2])

        copies = [start(0), start(1)]
        ms = [jnp.full((L,), NEG_INF, jnp.float32) for _ in range(G)]
        is_ = [jnp.zeros((L,), jnp.int32) for _ in range(G)]
        for j in range(NCH):
            buf = bufs[j % 2]
            copies[j % 2].wait()
            t0 = jnp.zeros((L,), jnp.int32) + (off0 + j * VW)

            def step(v, carry, buf=buf):
                st = list(carry)
                t_vec = st[2 * G]
                for g in range(G):
                    val = buf[v, pl.ds((c * G + g) * L, L)]
                    gt = val > st[g]
                    st[g] = jnp.where(gt, val, st[g])
                    st[G + g] = jnp.where(gt, t_vec, st[G + g])
                st[2 * G] = t_vec + 1
                return tuple(st)

            out_c = lax.fori_loop(0, VW, step, tuple(ms) + tuple(is_) + (t0,))
            ms, is_ = list(out_c[:G]), list(out_c[G:2 * G])
            if j + 2 < NCH:
                copies[j % 2] = start(j + 2)

        for g in range(G):
            mv[g, :] = ms[g]
            iv[g, :] = is_[g]
        for g in range(G):
            doff = pl.multiple_of(((c * G + g) * L + s) * L, 8)
            pltpu.sync_copy(mv.at[g], xchm_hbm.at[pl.ds(doff, L)])
            pltpu.sync_copy(iv.at[g], xchc_hbm.at[pl.ds(doff, L)])
        plsc.subcore_barrier()

        @pl.when(s < G)
        def _():
            poff = pl.multiple_of((c * G + s) * L * L, 8)
            pltpu.async_copy(xchm_hbm.at[pl.ds(poff, L * L)], gm, semC).wait()
            pltpu.async_copy(xchc_hbm.at[pl.ds(poff, L * L)], gi, semC).wait()
            m, idx = gm[pl.ds(0, L)], gi[pl.ds(0, L)]
            for t in range(1, L):
                m, idx = _merge(m, idx,
                                gm[pl.ds(t * L, L)], gi[pl.ds(t * L, L)])
            res_v[...] = idx
            resm_v[...] = m
            ooff = pl.multiple_of((c * G + s) * L, 8)
            pltpu.sync_copy(res_v, out_hbm.at[pl.ds(ooff, L)])
            pltpu.sync_copy(resm_v, outm_hbm.at[pl.ds(ooff, L)])

    res = body(xt)
    return res[0], res[1]


def _tc_argmax(xt):
    nb = (C - VS) // BV

    def tc_body(x_ref, io_ref, mo_ref, ms, is_):
        k = pl.program_id(0)

        @pl.when(k == 0)
        def _():
            ms[...] = jnp.full((8, 128), NEG_INF, jnp.float32)
            is_[...] = jnp.zeros((8, 128), jnp.int32)

        xb = x_ref[...]
        m = ms[...]
        idx = is_[...]
        rows = lax.broadcasted_iota(jnp.int32, (8, 128), 0) + (VS + k * BV)
        for r in range(BV // 8):
            blk = xb[r * 8:(r + 1) * 8, :]
            cur = rows + (r * 8)
            gt = blk > m
            m = jnp.where(gt, blk, m)
            idx = jnp.where(gt, cur, idx)
        ms[...] = m
        is_[...] = idx

        @pl.when(k == nb - 1)
        def _():
            mm, ii = m, idx
            for half in (4, 2, 1):
                ma, mb = mm[:half], mm[half:2 * half]
                ia, ib = ii[:half], ii[half:2 * half]
                take = (mb > ma) | ((mb == ma) & (ib < ia))
                mm = jnp.where(take, mb, ma)
                ii = jnp.where(take, ib, ia)
            io_ref[...] = ii
            mo_ref[...] = mm

    return pl.pallas_call(
        tc_body,
        grid=(nb,),
        in_specs=[pl.BlockSpec((BV, 128), lambda k: (VS // BV + k, 0))],
        out_specs=[pl.BlockSpec((1, 128), lambda k: (0, 0)),
                   pl.BlockSpec((1, 128), lambda k: (0, 0))],
        out_shape=[jax.ShapeDtypeStruct((1, 128), jnp.int32),
                   jax.ShapeDtypeStruct((1, 128), jnp.float32)],
        scratch_shapes=[pltpu.VMEM((8, 128), jnp.float32),
                        pltpu.VMEM((8, 128), jnp.int32)],
    )(xt)


def kernel(m_logits):
    xt = m_logits.T
    sc_idx, sc_m = _sc_argmax(xt)
    tc_idx, tc_m = _tc_argmax(xt)
    tc_idx, tc_m = tc_idx[0], tc_m[0]
    out = jnp.where(tc_m > sc_m, tc_idx, sc_idx)
    return out.reshape(R, 1).astype(jnp.int64)


# speedup vs baseline: 1.7022x; 1.7022x over previous
import jax
import jax.numpy as jnp
from jax import lax
from jax.experimental import pallas as pl
from jax.experimental.pallas import tpu as pltpu
from jax.experimental.pallas import tpu_sc as plsc

R, C = 128, 100000
L = 16
G = 4
VS = 28000
VW = 400
NCH = 5
WIN = VW * NCH
SHARD = VS // 16
BV = 1000
NEG_INF = float("-inf")


def _merge(ma, ia, mb, ib):
    take = (mb > ma) | ((mb == ma) & (ib < ia))
    return jnp.where(take, mb, ma), jnp.where(take, ib, ia)


def _sc_argmax(xt):
    mesh = plsc.VectorSubcoreMesh(core_axis_name="c", subcore_axis_name="s")

    @pl.kernel(
        out_type=(jax.ShapeDtypeStruct((R,), jnp.int32),
                  jax.ShapeDtypeStruct((R,), jnp.float32),
                  jax.ShapeDtypeStruct((2048,), jnp.float32),
                  jax.ShapeDtypeStruct((2048,), jnp.int32)),
        mesh=mesh,
        scratch_types=[
            pltpu.VMEM((VW, 128), jnp.float32),
            pltpu.VMEM((VW, 128), jnp.float32),
            pltpu.VMEM((G, L), jnp.float32),
            pltpu.VMEM((G, L), jnp.int32),
            pltpu.VMEM((L * L,), jnp.float32),
            pltpu.VMEM((L * L,), jnp.int32),
            pltpu.VMEM((L,), jnp.int32),
            pltpu.VMEM((L,), jnp.float32),
            pltpu.SemaphoreType.DMA,
            pltpu.SemaphoreType.DMA,
            pltpu.SemaphoreType.DMA,
        ],
    )
    def body(x_hbm, out_hbm, outm_hbm, xchm_hbm, xchc_hbm,
             bufA, bufB, mv, iv, gm, gi, res_v, resm_v, semA, semB, semC):
        s = lax.axis_index("s")
        c = lax.axis_index("c")
        bufs = (bufA, bufB)
        sems = (semA, semB)
        off0 = (s * SHARD) // 8 * 8
        off0 = pl.multiple_of(jnp.minimum(off0, VS - WIN), 8)

        def start(j):
            return pltpu.async_copy(
                x_hbm.at[pl.ds(pl.multiple_of(off0 + j * VW, 8), VW)],
                bufs[j % 2], sems[j % 2])

        copies = [start(0), start(1)]
        ms = [jnp.full((L,), NEG_INF, jnp.float32) for _ in range(G)]
        is_ = [jnp.zeros((L,), jnp.int32) for _ in range(G)]
        for j in range(NCH):
            buf = bufs[j % 2]
            copies[j % 2].wait()
            t0 = jnp.zeros((L,), jnp.int32) + (off0 + j * VW)

            def step(v, carry, buf=buf):
                st = list(carry)
                t_vec = st[2 * G]
                for g in range(G):
                    val = buf[v, pl.ds((c * G + g) * L, L)]
                    gt = val > st[g]
                    st[g] = jnp.where(gt, val, st[g])
                    st[G + g] = jnp.where(gt, t_vec, st[G + g])
                st[2 * G] = t_vec + 1
                return tuple(st)

            out_c = lax.fori_loop(0, VW, step, tuple(ms) + tuple(is_) + (t0,))
            ms, is_ = list(out_c[:G]), list(out_c[G:2 * G])
            if j + 2 < NCH:
                copies[j % 2] = start(j + 2)

        for g in range(G):
            mv[g, :] = ms[g]
            iv[g, :] = is_[g]
        for g in range(G):
            doff = pl.multiple_of(((c * G + g) * L + s) * L, 8)
            pltpu.sync_copy(mv.at[g], xchm_hbm.at[pl.ds(doff, L)])
            pltpu.sync_copy(iv.at[g], xchc_hbm.at[pl.ds(doff, L)])
        plsc.subcore_barrier()

        @pl.when(s < G)
        def _():
            poff = pl.multiple_of((c * G + s) * L * L, 8)
            pltpu.async_copy(xchm_hbm.at[pl.ds(poff, L * L)], gm, semC).wait()
            pltpu.async_copy(xchc_hbm.at[pl.ds(poff, L * L)], gi, semC).wait()
            m, idx = gm[pl.ds(0, L)], gi[pl.ds(0, L)]
            for t in range(1, L):
                m, idx = _merge(m, idx,
                                gm[pl.ds(t * L, L)], gi[pl.ds(t * L, L)])
            res_v[...] = idx
            resm_v[...] = m
            ooff = pl.multiple_of((c * G + s) * L, 8)
            pltpu.sync_copy(res_v, out_hbm.at[pl.ds(ooff, L)])
            pltpu.sync_copy(resm_v, outm_hbm.at[pl.ds(ooff, L)])

    res = body(xt)
    return res[0], res[1]


def _tc_argmax(xt):
    nb = (C - VS) // BV

    def tc_body(x_ref, io_ref, mo_ref, ms, is_):
        k = pl.program_id(0)

        @pl.when(k == 0)
        def _():
            ms[...] = jnp.full((1, 128), NEG_INF, jnp.float32)
            is_[...] = jnp.zeros((1, 128), jnp.int32)

        xb = x_ref[...]
        bm = jnp.max(xb, axis=0, keepdims=True)
        bi = jnp.argmax(xb, axis=0)[None].astype(jnp.int32)
        cur = bi + (VS + k * BV)
        m = ms[...]
        idx = is_[...]
        gt = bm > m
        ms[...] = jnp.where(gt, bm, m)
        is_[...] = jnp.where(gt, cur, idx)

        @pl.when(k == nb - 1)
        def _():
            io_ref[...] = is_[...]
            mo_ref[...] = ms[...]

    return pl.pallas_call(
        tc_body,
        grid=(nb,),
        in_specs=[pl.BlockSpec((BV, 128), lambda k: (VS // BV + k, 0))],
        out_specs=[pl.BlockSpec((1, 128), lambda k: (0, 0)),
                   pl.BlockSpec((1, 128), lambda k: (0, 0))],
        out_shape=[jax.ShapeDtypeStruct((1, 128), jnp.int32),
                   jax.ShapeDtypeStruct((1, 128), jnp.float32)],
        scratch_shapes=[pltpu.VMEM((1, 128), jnp.float32),
                        pltpu.VMEM((1, 128), jnp.int32)],
    )(xt)


def kernel(m_logits):
    xt = m_logits.T
    sc_idx, sc_m = _sc_argmax(xt)
    tc_idx, tc_m = _tc_argmax(xt)
    tc_idx, tc_m = tc_idx[0], tc_m[0]
    out = jnp.where(tc_m > sc_m, tc_idx, sc_idx)
    return out.reshape(R, 1).astype(jnp.int64)


# speedup vs baseline: 1.9597x; 1.1512x over previous
import jax
import jax.numpy as jnp
from jax import lax
from jax.experimental import pallas as pl
from jax.experimental.pallas import tpu as pltpu
from jax.experimental.pallas import tpu_sc as plsc

R, C = 128, 100000
L = 16
G = 4
VS = 46000
VW = 400
NCH = 8
WIN = VW * NCH
SHARD = VS // 16
BV = 1000
NEG_INF = float("-inf")


def _merge(ma, ia, mb, ib):
    take = (mb > ma) | ((mb == ma) & (ib < ia))
    return jnp.where(take, mb, ma), jnp.where(take, ib, ia)


def _sc_argmax(xt):
    mesh = plsc.VectorSubcoreMesh(core_axis_name="c", subcore_axis_name="s")

    @pl.kernel(
        out_type=(jax.ShapeDtypeStruct((R,), jnp.int32),
                  jax.ShapeDtypeStruct((R,), jnp.float32),
                  jax.ShapeDtypeStruct((2048,), jnp.float32),
                  jax.ShapeDtypeStruct((2048,), jnp.int32)),
        mesh=mesh,
        scratch_types=[
            pltpu.VMEM((VW, 128), jnp.float32),
            pltpu.VMEM((VW, 128), jnp.float32),
            pltpu.VMEM((G, L), jnp.float32),
            pltpu.VMEM((G, L), jnp.int32),
            pltpu.VMEM((L * L,), jnp.float32),
            pltpu.VMEM((L * L,), jnp.int32),
            pltpu.VMEM((L,), jnp.int32),
            pltpu.VMEM((L,), jnp.float32),
            pltpu.SemaphoreType.DMA,
            pltpu.SemaphoreType.DMA,
            pltpu.SemaphoreType.DMA,
        ],
    )
    def body(x_hbm, out_hbm, outm_hbm, xchm_hbm, xchc_hbm,
             bufA, bufB, mv, iv, gm, gi, res_v, resm_v, semA, semB, semC):
        s = lax.axis_index("s")
        c = lax.axis_index("c")
        bufs = (bufA, bufB)
        sems = (semA, semB)
        off0 = (s * SHARD) // 8 * 8
        off0 = pl.multiple_of(jnp.minimum(off0, VS - WIN), 8)

        def start(j):
            return pltpu.async_copy(
                x_hbm.at[pl.ds(pl.multiple_of(off0 + j * VW, 8), VW)],
                bufs[j % 2], sems[j % 2])

        copies = [start(0), start(1)]
        ms = [jnp.full((L,), NEG_INF, jnp.float32) for _ in range(G)]
        is_ = [jnp.zeros((L,), jnp.int32) for _ in range(G)]
        for j in range(NCH):
            buf = bufs[j % 2]
            copies[j % 2].wait()
            t0 = jnp.zeros((L,), jnp.int32) + (off0 + j * VW)

            def step(v, carry, buf=buf):
                st = list(carry)
                t_vec = st[2 * G]
                for g in range(G):
                    val = buf[v, pl.ds((c * G + g) * L, L)]
                    gt = val > st[g]
                    st[g] = jnp.where(gt, val, st[g])
                    st[G + g] = jnp.where(gt, t_vec, st[G + g])
                st[2 * G] = t_vec + 1
                return tuple(st)

            out_c = lax.fori_loop(0, VW, step, tuple(ms) + tuple(is_) + (t0,))
            ms, is_ = list(out_c[:G]), list(out_c[G:2 * G])
            if j + 2 < NCH:
                copies[j % 2] = start(j + 2)

        for g in range(G):
            mv[g, :] = ms[g]
            iv[g, :] = is_[g]
        for g in range(G):
            doff = pl.multiple_of(((c * G + g) * L + s) * L, 8)
            pltpu.sync_copy(mv.at[g], xchm_hbm.at[pl.ds(doff, L)])
            pltpu.sync_copy(iv.at[g], xchc_hbm.at[pl.ds(doff, L)])
        plsc.subcore_barrier()

        @pl.when(s < G)
        def _():
            poff = pl.multiple_of((c * G + s) * L * L, 8)
            pltpu.async_copy(xchm_hbm.at[pl.ds(poff, L * L)], gm, semC).wait()
            pltpu.async_copy(xchc_hbm.at[pl.ds(poff, L * L)], gi, semC).wait()
            m, idx = gm[pl.ds(0, L)], gi[pl.ds(0, L)]
            for t in range(1, L):
                m, idx = _merge(m, idx,
                                gm[pl.ds(t * L, L)], gi[pl.ds(t * L, L)])
            res_v[...] = idx
            resm_v[...] = m
            ooff = pl.multiple_of((c * G + s) * L, 8)
            pltpu.sync_copy(res_v, out_hbm.at[pl.ds(ooff, L)])
            pltpu.sync_copy(resm_v, outm_hbm.at[pl.ds(ooff, L)])

    res = body(xt)
    return res[0], res[1]


def _tc_argmax(xt):
    nb = (C - VS) // BV

    def tc_body(x_ref, io_ref, mo_ref, ms, is_):
        k = pl.program_id(0)

        @pl.when(k == 0)
        def _():
            ms[...] = jnp.full((1, 128), NEG_INF, jnp.float32)
            is_[...] = jnp.zeros((1, 128), jnp.int32)

        xb = x_ref[...]
        bm = jnp.max(xb, axis=0, keepdims=True)
        bi = jnp.argmax(xb, axis=0)[None].astype(jnp.int32)
        cur = bi + (VS + k * BV)
        m = ms[...]
        idx = is_[...]
        gt = bm > m
        ms[...] = jnp.where(gt, bm, m)
        is_[...] = jnp.where(gt, cur, idx)

        @pl.when(k == nb - 1)
        def _():
            io_ref[...] = is_[...]
            mo_ref[...] = ms[...]

    return pl.pallas_call(
        tc_body,
        grid=(nb,),
        in_specs=[pl.BlockSpec((BV, 128), lambda k: (VS // BV + k, 0))],
        out_specs=[pl.BlockSpec((1, 128), lambda k: (0, 0)),
                   pl.BlockSpec((1, 128), lambda k: (0, 0))],
        out_shape=[jax.ShapeDtypeStruct((1, 128), jnp.int32),
                   jax.ShapeDtypeStruct((1, 128), jnp.float32)],
        scratch_shapes=[pltpu.VMEM((1, 128), jnp.float32),
                        pltpu.VMEM((1, 128), jnp.int32)],
    )(xt)


def kernel(m_logits):
    xt = m_logits.T
    sc_idx, sc_m = _sc_argmax(xt)
    tc_idx, tc_m = _tc_argmax(xt)
    tc_idx, tc_m = tc_idx[0], tc_m[0]
    out = jnp.where(tc_m > sc_m, tc_idx, sc_idx)
    return out.reshape(R, 1).astype(jnp.int64)
